# Initial kernel scaffold; baseline (speedup 1.0000x reference)
import jax
import jax.numpy as jnp
from jax.experimental import pallas as pl


def kernel(x, edge_index0, edge_index1, edge_index2, ew0, ew1, ew2, W1, b1, W2, b2, W3, b3):
    raise NotImplementedError("write your pallas kernel here")


# speedup vs baseline: 13.7820x; 13.7820x over previous
import functools

import jax
import jax.numpy as jnp
from jax import lax
from jax.experimental import pallas as pl
from jax.experimental.pallas import tpu as pltpu
from jax.experimental.pallas import tpu_sc as plsc

f32 = jnp.float32
i32 = jnp.int32

NC = 2
NS = 16
NW = NC * NS
LANE = 16
K = 80
CHE = 2000

N = 10000
E = 320000
D = 128
NP = 10240

NCHUNK = N // K
NCHUNKP = NP // K
PER_W = E // NW
ROWS_PER_W = PER_W // K
_MESH = plsc.VectorSubcoreMesh(core_axis_name="c", subcore_axis_name="s")


def _deg_body(src0, dst0, ew0, src1, dst1, ew1, src2, dst2, ew2, out,
              sbig, dbig, wbig, sidx, didx, ones_v, zeros_v,
              do0, do1, do2, di0, di1, di2, ct0, ct1, ct2):
    cid = lax.axis_index("c")
    sid = lax.axis_index("s")
    wid = sid * NC + cid
    ones16 = jnp.ones((LANE,), f32)
    zeros16 = jnp.zeros((LANE,), f32)
    for t in range(K // LANE):
        ones_v[pl.ds(t * LANE, LANE)] = ones16
    for t in range(8 * K // LANE):
        zeros_v[pl.ds(t * LANE, LANE)] = zeros16
    sh = (do0, do1, do2, di0, di1, di2, ct0, ct1, ct2)

    for a in range(9):
        pltpu.sync_copy(zeros_v, sh[a].at[pl.ds(sid * 8 * K, 8 * K)])
    plsc.subcore_barrier()

    graphs = ((src0, dst0, ew0, do0, di0, ct0),
              (src1, dst1, ew1, do1, di1, ct1),
              (src2, dst2, ew2, do2, di2, ct2))
    for (sg, dg, wg, dog, dig, ctg) in graphs:
        def stage_body(st, _, sg=sg, dg=dg, wg=wg, dog=dog, dig=dig, ctg=ctg):
            base = wid * PER_W + st * CHE
            pltpu.sync_copy(sg.at[pl.ds(base, CHE)], sbig)
            pltpu.sync_copy(dg.at[pl.ds(base, CHE)], dbig)
            pltpu.sync_copy(wg.at[pl.ds(base, CHE)], wbig)

            def jbody(j, _):
                off = j * K
                for t in range(K // LANE):
                    sidx[pl.ds(t * LANE, LANE)] = (
                        sbig[pl.ds(off + t * LANE, LANE)])
                    didx[pl.ds(t * LANE, LANE)] = (
                        dbig[pl.ds(off + t * LANE, LANE)])
                pltpu.sync_copy(wbig.at[pl.ds(off, K)], dog.at[sidx],
                                add=True)
                pltpu.sync_copy(wbig.at[pl.ds(off, K)], dig.at[didx],
                                add=True)
                pltpu.sync_copy(ones_v, ctg.at[didx], add=True)
                return 0

            lax.fori_loop(0, CHE // K, jbody, 0)
            return 0

        lax.fori_loop(0, PER_W // CHE, stage_body, 0)
    plsc.subcore_barrier()

    for a in range(9):
        pltpu.sync_copy(sh[a].at[pl.ds(sid * 8 * K, 8 * K)],
                        out.at[cid, 0, pl.ds(a * NP + sid * 8 * K, 8 * K)])


_deg_call = functools.partial(
    pl.kernel, _deg_body,
    out_type=jax.ShapeDtypeStruct((NC, 1, 9 * NP), f32),
    mesh=_MESH,
    scratch_types=(
        [pltpu.VMEM((CHE,), i32), pltpu.VMEM((CHE,), i32),
         pltpu.VMEM((CHE,), f32),
         pltpu.VMEM((K,), i32), pltpu.VMEM((K,), i32),
         pltpu.VMEM((K,), f32), pltpu.VMEM((8 * K,), f32)]
        + [pltpu.VMEM_SHARED((NP,), f32)] * 9
    ),
)()


def _agg_body(hs, srcg, dstg, ewg, out, srcv, dstv, ewv, rows_v, agg_sh, sem):
    cid = lax.axis_index("c")
    sid = lax.axis_index("s")
    wid = sid * NC + cid
    zeros16 = jnp.zeros((LANE,), f32)

    def zrow(r, _):
        for sbl in range(D // LANE):
            rows_v[r, pl.ds(sbl * LANE, LANE)] = zeros16
        return 0

    lax.fori_loop(0, K, zrow, 0)

    def zbody(j, _):
        i = sid + NS * j

        @pl.when(i < NCHUNK)
        def _():
            pltpu.sync_copy(rows_v, agg_sh.at[pl.ds(i * K, K)])
        return 0

    lax.fori_loop(0, (NCHUNK + NS - 1) // NS, zbody, 0)
    plsc.subcore_barrier()

    def cbody(c, _):
        base = wid * PER_W + c * K
        pltpu.sync_copy(srcg.at[pl.ds(base, K)], srcv)
        pltpu.sync_copy(dstg.at[pl.ds(base, K)], dstv)
        pltpu.sync_copy(ewg.at[pl.ds(base, K)], ewv)
        pltpu.async_copy(hs.at[srcv], rows_v, sem).wait()

        def gbody(g, _):
            w16 = ewv[pl.ds(g * LANE, LANE)]
            for jj in range(LANE):
                wv = w16.at[jnp.full((LANE,), jj, i32)].get(
                    mode="promise_in_bounds")
                row = g * LANE + jj
                for sbl in range(D // LANE):
                    col = sbl * LANE
                    rows_v[row, pl.ds(col, LANE)] = (
                        rows_v[row, pl.ds(col, LANE)] * wv)
            return 0

        lax.fori_loop(0, K // LANE, gbody, 0)
        pltpu.sync_copy(rows_v, agg_sh.at[dstv], add=True)
        return 0

    lax.fori_loop(0, ROWS_PER_W, cbody, 0)
    plsc.subcore_barrier()

    def obody(j, _):
        i = sid + NS * j

        @pl.when(i < NCHUNK)
        def _():
            pltpu.sync_copy(agg_sh.at[pl.ds(i * K, K)],
                            out.at[cid, pl.ds(i * K, K)])
        return 0

    lax.fori_loop(0, (NCHUNK + NS - 1) // NS, obody, 0)


_agg_call = functools.partial(
    pl.kernel, _agg_body,
    out_type=jax.ShapeDtypeStruct((NC, N, D), f32),
    mesh=_MESH,
    scratch_types=[
        pltpu.VMEM((K,), i32), pltpu.VMEM((K,), i32), pltpu.VMEM((K,), f32),
        pltpu.VMEM((K, D), f32),
        pltpu.VMEM_SHARED((N, D), f32),
        pltpu.SemaphoreType.DMA,
    ],
)()


def _prep_body(degs_ref, rs_ref, scl_ref):
    d = degs_ref[0] + degs_ref[1]
    dout = d[0:3]
    din = d[3:6]
    cnt = d[6:9]
    rs_ref[...] = lax.rsqrt(jnp.where(dout > 0, dout, 1.0))
    scl_ref[...] = (lax.rsqrt(jnp.where(din > 0, din, 1.0))
                    / jnp.maximum(cnt, 1.0))


def _xscale_body(x_ref, rs_ref, out_ref):
    out_ref[...] = x_ref[...] * rs_ref[...]


def _layer_body(aggp_ref, scl_ref, rsn_ref, w_ref, b_ref, out_ref):
    a = aggp_ref[0] + aggp_ref[1]
    h = a * scl_ref[...]
    h = jnp.dot(h, w_ref[...], preferred_element_type=f32) + b_ref[...]
    out_ref[...] = h * rsn_ref[...]


_R = 2000


def _layer_call(aggp, scl_col, rsn_col, w, b_row):
    return pl.pallas_call(
        _layer_body,
        out_shape=jax.ShapeDtypeStruct((N, D), f32),
        grid=(N // _R,),
        in_specs=[
            pl.BlockSpec((NC, _R, D), lambda i: (0, i, 0)),
            pl.BlockSpec((_R, 1), lambda i: (i, 0)),
            pl.BlockSpec((_R, 1), lambda i: (i, 0)),
            pl.BlockSpec((D, D), lambda i: (0, 0)),
            pl.BlockSpec((1, D), lambda i: (0, 0)),
        ],
        out_specs=pl.BlockSpec((_R, D), lambda i: (i, 0)),
    )(aggp, scl_col, rsn_col, w, b_row)


def kernel(x, edge_index0, edge_index1, edge_index2, ew0, ew1, ew2,
           W1, b1, W2, b2, W3, b3):
    srcs = [ei[0] for ei in (edge_index0, edge_index1, edge_index2)]
    dsts = [ei[1] for ei in (edge_index0, edge_index1, edge_index2)]
    ews = (ew0, ew1, ew2)

    degs = _deg_call(srcs[0], dsts[0], ews[0],
                     srcs[1], dsts[1], ews[1],
                     srcs[2], dsts[2], ews[2])
    degs = degs.reshape(NC, 9, NP)

    rs3, scl3 = pl.pallas_call(
        _prep_body,
        out_shape=(jax.ShapeDtypeStruct((3, NP), f32),
                   jax.ShapeDtypeStruct((3, NP), f32)),
    )(degs)

    rs_cols = [rs3[g, :N].reshape(N, 1) for g in range(3)]
    scl_cols = [scl3[g, :N].reshape(N, 1) for g in range(3)]
    ones_col = jnp.ones((N, 1), f32)

    h = pl.pallas_call(
        _xscale_body,
        out_shape=jax.ShapeDtypeStruct((N, D), f32),
    )(x, rs_cols[0])

    weights = ((W1, b1), (W2, b2), (W3, b3))
    for i, (w, b) in enumerate(weights):
        aggp = _agg_call(h, srcs[i], dsts[i], ews[i])
        rsn = rs_cols[i + 1] if i < 2 else ones_col
        h = _layer_call(aggp, scl_cols[i], rsn, w, b.reshape(1, D))
    return h


# speedup vs baseline: 23.6394x; 1.7152x over previous
import functools

import jax
import jax.numpy as jnp
from jax import lax
from jax.experimental import pallas as pl
from jax.experimental.pallas import tpu as pltpu
from jax.experimental.pallas import tpu_sc as plsc

f32 = jnp.float32
i32 = jnp.int32

NC = 2
NS = 16
NW = NC * NS
LANE = 16
K = 80
CHE = 2000

N = 10000
E = 320000
D = 128
NP = 10240

NCHUNK = N // K
NCHUNKP = NP // K
PER_W = E // NW
ROWS_PER_W = PER_W // K
_MESH = plsc.VectorSubcoreMesh(core_axis_name="c", subcore_axis_name="s")


def _deg_body(src0, dst0, ew0, src1, dst1, ew1, src2, dst2, ew2, out,
              sbig, dbig, wbig, sidx, didx, ones_v, zeros_v,
              do0, do1, do2, di0, di1, di2, ct0, ct1, ct2):
    cid = lax.axis_index("c")
    sid = lax.axis_index("s")
    wid = sid * NC + cid
    ones16 = jnp.ones((LANE,), f32)
    zeros16 = jnp.zeros((LANE,), f32)
    for t in range(K // LANE):
        ones_v[pl.ds(t * LANE, LANE)] = ones16
    for t in range(8 * K // LANE):
        zeros_v[pl.ds(t * LANE, LANE)] = zeros16
    sh = (do0, do1, do2, di0, di1, di2, ct0, ct1, ct2)

    for a in range(9):
        pltpu.sync_copy(zeros_v, sh[a].at[pl.ds(sid * 8 * K, 8 * K)])
    plsc.subcore_barrier()

    graphs = ((src0, dst0, ew0, do0, di0, ct0),
              (src1, dst1, ew1, do1, di1, ct1),
              (src2, dst2, ew2, do2, di2, ct2))
    for (sg, dg, wg, dog, dig, ctg) in graphs:
        def stage_body(st, _, sg=sg, dg=dg, wg=wg, dog=dog, dig=dig, ctg=ctg):
            base = wid * PER_W + st * CHE
            pltpu.sync_copy(sg.at[pl.ds(base, CHE)], sbig)
            pltpu.sync_copy(dg.at[pl.ds(base, CHE)], dbig)
            pltpu.sync_copy(wg.at[pl.ds(base, CHE)], wbig)

            def jbody(j, _):
                off = j * K
                for t in range(K // LANE):
                    sidx[pl.ds(t * LANE, LANE)] = (
                        sbig[pl.ds(off + t * LANE, LANE)])
                    didx[pl.ds(t * LANE, LANE)] = (
                        dbig[pl.ds(off + t * LANE, LANE)])
                pltpu.sync_copy(wbig.at[pl.ds(off, K)], dog.at[sidx],
                                add=True)
                pltpu.sync_copy(wbig.at[pl.ds(off, K)], dig.at[didx],
                                add=True)
                pltpu.sync_copy(ones_v, ctg.at[didx], add=True)
                return 0

            lax.fori_loop(0, CHE // K, jbody, 0)
            return 0

        lax.fori_loop(0, PER_W // CHE, stage_body, 0)
    plsc.subcore_barrier()

    for a in range(9):
        pltpu.sync_copy(sh[a].at[pl.ds(sid * 8 * K, 8 * K)],
                        out.at[cid, 0, pl.ds(a * NP + sid * 8 * K, 8 * K)])


_deg_call = functools.partial(
    pl.kernel, _deg_body,
    out_type=jax.ShapeDtypeStruct((NC, 1, 9 * NP), f32),
    mesh=_MESH,
    scratch_types=(
        [pltpu.VMEM((CHE,), i32), pltpu.VMEM((CHE,), i32),
         pltpu.VMEM((CHE,), f32),
         pltpu.VMEM((K,), i32), pltpu.VMEM((K,), i32),
         pltpu.VMEM((K,), f32), pltpu.VMEM((8 * K,), f32)]
        + [pltpu.VMEM_SHARED((NP,), f32)] * 9
    ),
)()


def _agg_body(hs, srcg, dstg, ewg, out,
              sbig, wbig, didx2, rows2, agg_sh,
              gsem0, gsem1, ssem0, ssem1, dsem0, dsem1):
    cid = lax.axis_index("c")
    sid = lax.axis_index("s")
    wid = sid * NC + cid
    base_w = wid * PER_W
    zeros16 = jnp.zeros((LANE,), f32)

    pltpu.sync_copy(srcg.at[pl.ds(base_w, PER_W)], sbig)
    pltpu.sync_copy(ewg.at[pl.ds(base_w, PER_W)], wbig)

    def zrow(r, _):
        for sbl in range(D // LANE):
            rows2[0, r, pl.ds(sbl * LANE, LANE)] = zeros16
        return 0

    lax.fori_loop(0, K, zrow, 0)

    def zbody(j, _):
        i = sid + NS * j

        @pl.when(i < NCHUNK)
        def _():
            pltpu.sync_copy(rows2.at[0], agg_sh.at[pl.ds(i * K, K)])
        return 0

    lax.fori_loop(0, (NCHUNK + NS - 1) // NS, zbody, 0)
    plsc.subcore_barrier()

    gsems = (gsem0, gsem1)
    ssems = (ssem0, ssem1)
    dsems = (dsem0, dsem1)

    def fire_didx(slot, c):
        pltpu.async_copy(dstg.at[pl.ds(base_w + c * K, K)], didx2.at[slot],
                         dsems[slot])

    def wait_didx(slot, c):
        pltpu.make_async_copy(dstg.at[pl.ds(base_w + c * K, K)],
                              didx2.at[slot], dsems[slot]).wait()

    def fire_gather(slot, c):
        pltpu.async_copy(hs.at[sbig.at[pl.ds(c * K, K)]], rows2.at[slot],
                         gsems[slot])

    def wait_gather(slot, c):
        pltpu.make_async_copy(hs.at[sbig.at[pl.ds(c * K, K)]],
                              rows2.at[slot], gsems[slot]).wait()

    def fire_scatter(slot):
        pltpu.async_copy(rows2.at[slot], agg_sh.at[didx2.at[slot]],
                         ssems[slot], add=True)

    def wait_scatter(slot):
        pltpu.make_async_copy(rows2.at[slot], agg_sh.at[didx2.at[slot]],
                              ssems[slot]).wait()

    def scale(slot, c):
        def gbody(g, _):
            w16 = wbig[pl.ds(c * K + g * LANE, LANE)]
            for jj in range(LANE):
                wv = w16.at[jnp.full((LANE,), jj, i32)].get(
                    mode="promise_in_bounds")
                row = g * LANE + jj
                for sbl in range(D // LANE):
                    col = sbl * LANE
                    rows2[slot, row, pl.ds(col, LANE)] = (
                        rows2[slot, row, pl.ds(col, LANE)] * wv)
            return 0

        lax.fori_loop(0, K // LANE, gbody, 0)

    fire_didx(0, 0)
    fire_gather(0, 0)
    wait_gather(0, 0)
    scale(0, 0)
    wait_didx(0, 0)
    fire_scatter(0)
    fire_didx(1, 1)
    fire_gather(1, 1)

    def pair(cc, _):
        c1 = 2 * cc + 1
        wait_gather(1, c1)
        scale(1, c1)
        wait_didx(1, c1)
        fire_scatter(1)
        wait_scatter(0)
        fire_didx(0, c1 + 1)
        fire_gather(0, c1 + 1)

        c2 = c1 + 1
        wait_gather(0, c2)
        scale(0, c2)
        wait_didx(0, c2)
        fire_scatter(0)
        wait_scatter(1)

        @pl.when(c2 < ROWS_PER_W - 1)
        def _():
            fire_didx(1, c2 + 1)
            fire_gather(1, c2 + 1)
        return 0

    lax.fori_loop(0, (ROWS_PER_W - 1) // 2, pair, 0)
    wait_scatter(0)
    plsc.subcore_barrier()

    def obody(j, _):
        i = sid + NS * j

        @pl.when(i < NCHUNK)
        def _():
            pltpu.sync_copy(agg_sh.at[pl.ds(i * K, K)],
                            out.at[cid, pl.ds(i * K, K)])
        return 0

    lax.fori_loop(0, (NCHUNK + NS - 1) // NS, obody, 0)


_agg_call = functools.partial(
    pl.kernel, _agg_body,
    out_type=jax.ShapeDtypeStruct((NC, N, D), f32),
    mesh=_MESH,
    scratch_types=[
        pltpu.VMEM((PER_W,), i32), pltpu.VMEM((PER_W,), f32),
        pltpu.VMEM((2, K), i32),
        pltpu.VMEM((2, K, D), f32),
        pltpu.VMEM_SHARED((N, D), f32),
        pltpu.SemaphoreType.DMA, pltpu.SemaphoreType.DMA,
        pltpu.SemaphoreType.DMA, pltpu.SemaphoreType.DMA,
        pltpu.SemaphoreType.DMA, pltpu.SemaphoreType.DMA,
    ],
)()


def _prep_body(degs_ref, rs_ref, scl_ref):
    d = degs_ref[0] + degs_ref[1]
    dout = d[0:3]
    din = d[3:6]
    cnt = d[6:9]
    rs_ref[...] = lax.rsqrt(jnp.where(dout > 0, dout, 1.0))
    scl_ref[...] = (lax.rsqrt(jnp.where(din > 0, din, 1.0))
                    / jnp.maximum(cnt, 1.0))


def _xscale_body(x_ref, rs_ref, out_ref):
    out_ref[...] = x_ref[...] * rs_ref[...]


def _layer_body(aggp_ref, scl_ref, rsn_ref, w_ref, b_ref, out_ref):
    a = aggp_ref[0] + aggp_ref[1]
    h = a * scl_ref[...]
    h = jnp.dot(h, w_ref[...], preferred_element_type=f32) + b_ref[...]
    out_ref[...] = h * rsn_ref[...]


_R = 2000


def _layer_call(aggp, scl_col, rsn_col, w, b_row):
    return pl.pallas_call(
        _layer_body,
        out_shape=jax.ShapeDtypeStruct((N, D), f32),
        grid=(N // _R,),
        in_specs=[
            pl.BlockSpec((NC, _R, D), lambda i: (0, i, 0)),
            pl.BlockSpec((_R, 1), lambda i: (i, 0)),
            pl.BlockSpec((_R, 1), lambda i: (i, 0)),
            pl.BlockSpec((D, D), lambda i: (0, 0)),
            pl.BlockSpec((1, D), lambda i: (0, 0)),
        ],
        out_specs=pl.BlockSpec((_R, D), lambda i: (i, 0)),
    )(aggp, scl_col, rsn_col, w, b_row)


def kernel(x, edge_index0, edge_index1, edge_index2, ew0, ew1, ew2,
           W1, b1, W2, b2, W3, b3):
    srcs = [ei[0] for ei in (edge_index0, edge_index1, edge_index2)]
    dsts = [ei[1] for ei in (edge_index0, edge_index1, edge_index2)]
    ews = (ew0, ew1, ew2)

    degs = _deg_call(srcs[0], dsts[0], ews[0],
                     srcs[1], dsts[1], ews[1],
                     srcs[2], dsts[2], ews[2])
    degs = degs.reshape(NC, 9, NP)

    rs3, scl3 = pl.pallas_call(
        _prep_body,
        out_shape=(jax.ShapeDtypeStruct((3, NP), f32),
                   jax.ShapeDtypeStruct((3, NP), f32)),
    )(degs)

    rs_cols = [rs3[g, :N].reshape(N, 1) for g in range(3)]
    scl_cols = [scl3[g, :N].reshape(N, 1) for g in range(3)]
    ones_col = jnp.ones((N, 1), f32)

    h = pl.pallas_call(
        _xscale_body,
        out_shape=jax.ShapeDtypeStruct((N, D), f32),
    )(x, rs_cols[0])

    weights = ((W1, b1), (W2, b2), (W3, b3))
    for i, (w, b) in enumerate(weights):
        aggp = _agg_call(h, srcs[i], dsts[i], ews[i])
        rsn = rs_cols[i + 1] if i < 2 else ones_col
        h = _layer_call(aggp, scl_cols[i], rsn, w, b.reshape(1, D))
    return h


# speedup vs baseline: 28.3267x; 1.1983x over previous
import functools

import jax
import jax.numpy as jnp
from jax import lax
from jax.experimental import pallas as pl
from jax.experimental.pallas import tpu as pltpu
from jax.experimental.pallas import tpu_sc as plsc

f32 = jnp.float32
i32 = jnp.int32

NC = 2
NS = 16
NW = NC * NS
LANE = 16
K = 80
CHE = 2000

N = 10000
E = 320000
D = 128
NP = 10240

NCHUNK = N // K
NCHUNKP = NP // K
PER_W = E // NW
ROWS_PER_W = PER_W // K
_MESH = plsc.VectorSubcoreMesh(core_axis_name="c", subcore_axis_name="s")


def _deg_body(src0, dst0, ew0, src1, dst1, ew1, src2, dst2, ew2, out,
              sbig, dbig, wbig, sidx, didx, ones_v, zeros_v,
              do0, do1, do2, di0, di1, di2, ct0, ct1, ct2):
    cid = lax.axis_index("c")
    sid = lax.axis_index("s")
    wid = sid * NC + cid
    ones16 = jnp.ones((LANE,), f32)
    zeros16 = jnp.zeros((LANE,), f32)
    for t in range(K // LANE):
        ones_v[pl.ds(t * LANE, LANE)] = ones16
    for t in range(8 * K // LANE):
        zeros_v[pl.ds(t * LANE, LANE)] = zeros16
    sh = (do0, do1, do2, di0, di1, di2, ct0, ct1, ct2)

    for a in range(9):
        pltpu.sync_copy(zeros_v, sh[a].at[pl.ds(sid * 8 * K, 8 * K)])
    plsc.subcore_barrier()

    graphs = ((src0, dst0, ew0, do0, di0, ct0),
              (src1, dst1, ew1, do1, di1, ct1),
              (src2, dst2, ew2, do2, di2, ct2))
    for (sg, dg, wg, dog, dig, ctg) in graphs:
        def stage_body(st, _, sg=sg, dg=dg, wg=wg, dog=dog, dig=dig, ctg=ctg):
            base = wid * PER_W + st * CHE
            pltpu.sync_copy(sg.at[pl.ds(base, CHE)], sbig)
            pltpu.sync_copy(dg.at[pl.ds(base, CHE)], dbig)
            pltpu.sync_copy(wg.at[pl.ds(base, CHE)], wbig)

            def jbody(j, _):
                off = j * K
                for t in range(K // LANE):
                    sidx[pl.ds(t * LANE, LANE)] = (
                        sbig[pl.ds(off + t * LANE, LANE)])
                    didx[pl.ds(t * LANE, LANE)] = (
                        dbig[pl.ds(off + t * LANE, LANE)])
                pltpu.sync_copy(wbig.at[pl.ds(off, K)], dog.at[sidx],
                                add=True)
                pltpu.sync_copy(wbig.at[pl.ds(off, K)], dig.at[didx],
                                add=True)
                pltpu.sync_copy(ones_v, ctg.at[didx], add=True)
                return 0

            lax.fori_loop(0, CHE // K, jbody, 0)
            return 0

        lax.fori_loop(0, PER_W // CHE, stage_body, 0)
    plsc.subcore_barrier()

    for a in range(9):
        pltpu.sync_copy(sh[a].at[pl.ds(sid * 8 * K, 8 * K)],
                        out.at[cid, 0, pl.ds(a * NP + sid * 8 * K, 8 * K)])


_deg_call = functools.partial(
    pl.kernel, _deg_body,
    out_type=jax.ShapeDtypeStruct((NC, 1, 9 * NP), f32),
    mesh=_MESH,
    scratch_types=(
        [pltpu.VMEM((CHE,), i32), pltpu.VMEM((CHE,), i32),
         pltpu.VMEM((CHE,), f32),
         pltpu.VMEM((K,), i32), pltpu.VMEM((K,), i32),
         pltpu.VMEM((K,), f32), pltpu.VMEM((8 * K,), f32)]
        + [pltpu.VMEM_SHARED((NP,), f32)] * 9
    ),
)()


def _agg_body(hs, srcg, dstg, ewg, out,
              sbig, wbig, didx2, rows2, agg_sh,
              gsem0, gsem1, ssem0, ssem1, dsem0, dsem1):
    cid = lax.axis_index("c")
    sid = lax.axis_index("s")
    wid = sid * NC + cid
    base_w = wid * PER_W
    zeros16 = jnp.zeros((LANE,), f32)

    pltpu.sync_copy(srcg.at[pl.ds(base_w, PER_W)], sbig)
    pltpu.sync_copy(ewg.at[pl.ds(base_w, PER_W)], wbig)

    def zrow(r, _):
        for sbl in range(D // LANE):
            rows2[0, r, pl.ds(sbl * LANE, LANE)] = zeros16
        return 0

    lax.fori_loop(0, K, zrow, 0)

    def zbody(j, _):
        i = sid + NS * j

        @pl.when(i < NCHUNK)
        def _():
            pltpu.sync_copy(rows2.at[0], agg_sh.at[pl.ds(i * K, K)])
        return 0

    lax.fori_loop(0, (NCHUNK + NS - 1) // NS, zbody, 0)
    plsc.subcore_barrier()

    gsems = (gsem0, gsem1)
    ssems = (ssem0, ssem1)
    dsems = (dsem0, dsem1)

    def fire_didx(slot, c):
        pltpu.async_copy(dstg.at[pl.ds(base_w + c * K, K)], didx2.at[slot],
                         dsems[slot])

    def wait_didx(slot, c):
        pltpu.make_async_copy(dstg.at[pl.ds(base_w + c * K, K)],
                              didx2.at[slot], dsems[slot]).wait()

    def fire_gather(slot, c):
        pltpu.async_copy(hs.at[sbig.at[pl.ds(c * K, K)]], rows2.at[slot],
                         gsems[slot])

    def wait_gather(slot, c):
        pltpu.make_async_copy(hs.at[sbig.at[pl.ds(c * K, K)]],
                              rows2.at[slot], gsems[slot]).wait()

    def fire_scatter(slot):
        pltpu.async_copy(rows2.at[slot], agg_sh.at[didx2.at[slot]],
                         ssems[slot], add=True)

    def wait_scatter(slot):
        pltpu.make_async_copy(rows2.at[slot], agg_sh.at[didx2.at[slot]],
                              ssems[slot]).wait()

    def scale(slot, c):
        def gbody(g, _):
            w16 = wbig[pl.ds(c * K + g * LANE, LANE)]
            for jj in range(LANE):
                wv = w16.at[jnp.full((LANE,), jj, i32)].get(
                    mode="promise_in_bounds")
                row = g * LANE + jj
                for sbl in range(D // LANE):
                    col = sbl * LANE
                    rows2[slot, row, pl.ds(col, LANE)] = (
                        rows2[slot, row, pl.ds(col, LANE)] * wv)
            return 0

        lax.fori_loop(0, K // LANE, gbody, 0)

    fire_didx(0, 0)
    fire_gather(0, 0)
    wait_gather(0, 0)
    fire_didx(1, 1)
    fire_gather(1, 1)
    scale(0, 0)
    wait_didx(0, 0)
    fire_scatter(0)

    def pair(cc, _):
        c1 = 2 * cc + 1
        wait_gather(1, c1)
        wait_scatter(0)
        fire_didx(0, c1 + 1)
        fire_gather(0, c1 + 1)
        scale(1, c1)
        wait_didx(1, c1)
        fire_scatter(1)

        c2 = c1 + 1
        wait_gather(0, c2)
        wait_scatter(1)

        @pl.when(c2 < ROWS_PER_W - 1)
        def _():
            fire_didx(1, c2 + 1)
            fire_gather(1, c2 + 1)
        scale(0, c2)
        wait_didx(0, c2)
        fire_scatter(0)
        return 0

    lax.fori_loop(0, (ROWS_PER_W - 1) // 2, pair, 0)
    wait_scatter(0)
    plsc.subcore_barrier()

    def obody(j, _):
        i = sid + NS * j

        @pl.when(i < NCHUNK)
        def _():
            pltpu.sync_copy(agg_sh.at[pl.ds(i * K, K)],
                            out.at[cid, pl.ds(i * K, K)])
        return 0

    lax.fori_loop(0, (NCHUNK + NS - 1) // NS, obody, 0)


_agg_call = functools.partial(
    pl.kernel, _agg_body,
    out_type=jax.ShapeDtypeStruct((NC, N, D), f32),
    mesh=_MESH,
    scratch_types=[
        pltpu.VMEM((PER_W,), i32), pltpu.VMEM((PER_W,), f32),
        pltpu.VMEM((2, K), i32),
        pltpu.VMEM((2, K, D), f32),
        pltpu.VMEM_SHARED((N, D), f32),
        pltpu.SemaphoreType.DMA, pltpu.SemaphoreType.DMA,
        pltpu.SemaphoreType.DMA, pltpu.SemaphoreType.DMA,
        pltpu.SemaphoreType.DMA, pltpu.SemaphoreType.DMA,
    ],
)()


def _prep_body(degs_ref, rs_ref, scl_ref):
    d = degs_ref[0] + degs_ref[1]
    dout = d[0:3]
    din = d[3:6]
    cnt = d[6:9]
    rs_ref[...] = lax.rsqrt(jnp.where(dout > 0, dout, 1.0))
    scl_ref[...] = (lax.rsqrt(jnp.where(din > 0, din, 1.0))
                    / jnp.maximum(cnt, 1.0))


def _xscale_body(x_ref, rs_ref, out_ref):
    out_ref[...] = x_ref[...] * rs_ref[...]


def _layer_body(aggp_ref, scl_ref, rsn_ref, w_ref, b_ref, out_ref):
    a = aggp_ref[0] + aggp_ref[1]
    h = a * scl_ref[...]
    h = jnp.dot(h, w_ref[...], preferred_element_type=f32) + b_ref[...]
    out_ref[...] = h * rsn_ref[...]


_R = 2000


def _layer_call(aggp, scl_col, rsn_col, w, b_row):
    return pl.pallas_call(
        _layer_body,
        out_shape=jax.ShapeDtypeStruct((N, D), f32),
        grid=(N // _R,),
        in_specs=[
            pl.BlockSpec((NC, _R, D), lambda i: (0, i, 0)),
            pl.BlockSpec((_R, 1), lambda i: (i, 0)),
            pl.BlockSpec((_R, 1), lambda i: (i, 0)),
            pl.BlockSpec((D, D), lambda i: (0, 0)),
            pl.BlockSpec((1, D), lambda i: (0, 0)),
        ],
        out_specs=pl.BlockSpec((_R, D), lambda i: (i, 0)),
    )(aggp, scl_col, rsn_col, w, b_row)


def kernel(x, edge_index0, edge_index1, edge_index2, ew0, ew1, ew2,
           W1, b1, W2, b2, W3, b3):
    srcs = [ei[0] for ei in (edge_index0, edge_index1, edge_index2)]
    dsts = [ei[1] for ei in (edge_index0, edge_index1, edge_index2)]
    ews = (ew0, ew1, ew2)

    degs = _deg_call(srcs[0], dsts[0], ews[0],
                     srcs[1], dsts[1], ews[1],
                     srcs[2], dsts[2], ews[2])
    degs = degs.reshape(NC, 9, NP)

    rs3, scl3 = pl.pallas_call(
        _prep_body,
        out_shape=(jax.ShapeDtypeStruct((3, NP), f32),
                   jax.ShapeDtypeStruct((3, NP), f32)),
    )(degs)

    rs_cols = [rs3[g, :N].reshape(N, 1) for g in range(3)]
    scl_cols = [scl3[g, :N].reshape(N, 1) for g in range(3)]
    ones_col = jnp.ones((N, 1), f32)

    h = pl.pallas_call(
        _xscale_body,
        out_shape=jax.ShapeDtypeStruct((N, D), f32),
    )(x, rs_cols[0])

    weights = ((W1, b1), (W2, b2), (W3, b3))
    for i, (w, b) in enumerate(weights):
        aggp = _agg_call(h, srcs[i], dsts[i], ews[i])
        rsn = rs_cols[i + 1] if i < 2 else ones_col
        h = _layer_call(aggp, scl_cols[i], rsn, w, b.reshape(1, D))
    return h


# speedup vs baseline: 28.9765x; 1.0229x over previous
import functools

import jax
import jax.numpy as jnp
from jax import lax
from jax.experimental import pallas as pl
from jax.experimental.pallas import tpu as pltpu
from jax.experimental.pallas import tpu_sc as plsc

f32 = jnp.float32
i32 = jnp.int32

NC = 2
NS = 16
NW = NC * NS
LANE = 16
K = 80
CHE = 2000

N = 10000
E = 320000
D = 128
NP = 10240

NCHUNK = N // K
NCHUNKP = NP // K
PER_W = E // NW
ROWS_PER_W = PER_W // K
_MESH = plsc.VectorSubcoreMesh(core_axis_name="c", subcore_axis_name="s")


KA = 128
NA = PER_W // KA
KT = PER_W - NA * KA


def _deg_body(src0, dst0, ew0, src1, dst1, ew1, src2, dst2, ew2, out,
              wbig, sidx2, didx2, sidx_t, didx_t, ones_v, zeros_v,
              do0, do1, do2, di0, di1, di2, ct0, ct1, ct2,
              isem0, isem1, asem0, asem1):
    cid = lax.axis_index("c")
    sid = lax.axis_index("s")
    wid = sid * NC + cid
    base_w = wid * PER_W
    ones16 = jnp.ones((LANE,), f32)
    zeros16 = jnp.zeros((LANE,), f32)
    for t in range(KA // LANE):
        ones_v[pl.ds(t * LANE, LANE)] = ones16
    for t in range(8 * K // LANE):
        zeros_v[pl.ds(t * LANE, LANE)] = zeros16
    sh = (do0, do1, do2, di0, di1, di2, ct0, ct1, ct2)
    isems = (isem0, isem1)
    asems = (asem0, asem1)

    for a in range(9):
        pltpu.sync_copy(zeros_v, sh[a].at[pl.ds(sid * 8 * K, 8 * K)])
    plsc.subcore_barrier()

    graphs = ((src0, dst0, ew0, do0, di0, ct0),
              (src1, dst1, ew1, do1, di1, ct1),
              (src2, dst2, ew2, do2, di2, ct2))
    for (sg, dg, wg, dog, dig, ctg) in graphs:
        pltpu.sync_copy(wg.at[pl.ds(base_w, PER_W)], wbig)

        def fire_idx(slot, c, sg=sg, dg=dg):
            pltpu.async_copy(sg.at[pl.ds(base_w + c * KA, KA)],
                             sidx2.at[slot], isems[slot])
            pltpu.async_copy(dg.at[pl.ds(base_w + c * KA, KA)],
                             didx2.at[slot], isems[slot])

        def wait_idx(slot, c, sg=sg, dg=dg):
            pltpu.make_async_copy(sg.at[pl.ds(base_w + c * KA, KA)],
                                  sidx2.at[slot], isems[slot]).wait()
            pltpu.make_async_copy(dg.at[pl.ds(base_w + c * KA, KA)],
                                  didx2.at[slot], isems[slot]).wait()

        def fire_scats(slot, c, dog=dog, dig=dig, ctg=ctg):
            w = wbig.at[pl.ds(c * KA, KA)]
            pltpu.async_copy(w, dog.at[sidx2.at[slot]], asems[slot],
                             add=True)
            pltpu.async_copy(w, dig.at[didx2.at[slot]], asems[slot],
                             add=True)
            pltpu.async_copy(ones_v, ctg.at[didx2.at[slot]], asems[slot],
                             add=True)

        def wait_scats(slot, dog=dog, dig=dig, ctg=ctg):
            pltpu.make_async_copy(wbig.at[pl.ds(0, KA)],
                                  dog.at[sidx2.at[slot]],
                                  asems[slot]).wait()
            pltpu.make_async_copy(wbig.at[pl.ds(0, KA)],
                                  dig.at[didx2.at[slot]],
                                  asems[slot]).wait()
            pltpu.make_async_copy(ones_v, ctg.at[didx2.at[slot]],
                                  asems[slot]).wait()

        fire_idx(0, 0)
        wait_idx(0, 0)
        fire_idx(1, 1)
        fire_scats(0, 0)

        def pair(cc, _, fire_idx=fire_idx, wait_idx=wait_idx,
                 fire_scats=fire_scats, wait_scats=wait_scats):
            c1 = 2 * cc + 1
            wait_idx(1, c1)
            wait_scats(0)
            fire_idx(0, c1 + 1)
            fire_scats(1, c1)

            c2 = c1 + 1
            wait_idx(0, c2)
            wait_scats(1)

            @pl.when(c2 < NA - 1)
            def _():
                fire_idx(1, c2 + 1)
            fire_scats(0, c2)
            return 0

        lax.fori_loop(0, (NA - 1) // 2, pair, 0)
        cl = NA - 1
        wait_idx(1, cl)
        wait_scats(0)
        fire_scats(1, cl)
        wait_scats(1)

        toff = base_w + NA * KA
        pltpu.sync_copy(sg.at[pl.ds(toff, KT)], sidx_t)
        pltpu.sync_copy(dg.at[pl.ds(toff, KT)], didx_t)
        pltpu.sync_copy(wbig.at[pl.ds(NA * KA, KT)], dog.at[sidx_t],
                        add=True)
        pltpu.sync_copy(wbig.at[pl.ds(NA * KA, KT)], dig.at[didx_t],
                        add=True)
        pltpu.sync_copy(ones_v.at[pl.ds(0, KT)], ctg.at[didx_t], add=True)
    plsc.subcore_barrier()

    for a in range(9):
        pltpu.sync_copy(sh[a].at[pl.ds(sid * 8 * K, 8 * K)],
                        out.at[cid, 0, pl.ds(a * NP + sid * 8 * K, 8 * K)])


_deg_call = functools.partial(
    pl.kernel, _deg_body,
    out_type=jax.ShapeDtypeStruct((NC, 1, 9 * NP), f32),
    mesh=_MESH,
    scratch_types=(
        [pltpu.VMEM((PER_W,), f32),
         pltpu.VMEM((2, KA), i32), pltpu.VMEM((2, KA), i32),
         pltpu.VMEM((KT,), i32), pltpu.VMEM((KT,), i32),
         pltpu.VMEM((KA,), f32), pltpu.VMEM((8 * K,), f32)]
        + [pltpu.VMEM_SHARED((NP,), f32)] * 9
        + [pltpu.SemaphoreType.DMA] * 4
    ),
)()


def _agg_body(hs, srcg, dstg, ewg, out,
              sbig, wbig, didx2, rows2, agg_sh,
              gsem0, gsem1, ssem0, ssem1, dsem0, dsem1):
    cid = lax.axis_index("c")
    sid = lax.axis_index("s")
    wid = sid * NC + cid
    base_w = wid * PER_W
    zeros16 = jnp.zeros((LANE,), f32)

    pltpu.sync_copy(srcg.at[pl.ds(base_w, PER_W)], sbig)
    pltpu.sync_copy(ewg.at[pl.ds(base_w, PER_W)], wbig)

    def zrow(r, _):
        for sbl in range(D // LANE):
            rows2[0, r, pl.ds(sbl * LANE, LANE)] = zeros16
        return 0

    lax.fori_loop(0, K, zrow, 0)

    def zbody(j, _):
        i = sid + NS * j

        @pl.when(i < NCHUNK)
        def _():
            pltpu.sync_copy(rows2.at[0], agg_sh.at[pl.ds(i * K, K)])
        return 0

    lax.fori_loop(0, (NCHUNK + NS - 1) // NS, zbody, 0)
    plsc.subcore_barrier()

    gsems = (gsem0, gsem1)
    ssems = (ssem0, ssem1)
    dsems = (dsem0, dsem1)

    def fire_didx(slot, c):
        pltpu.async_copy(dstg.at[pl.ds(base_w + c * K, K)], didx2.at[slot],
                         dsems[slot])

    def wait_didx(slot, c):
        pltpu.make_async_copy(dstg.at[pl.ds(base_w + c * K, K)],
                              didx2.at[slot], dsems[slot]).wait()

    def fire_gather(slot, c):
        pltpu.async_copy(hs.at[sbig.at[pl.ds(c * K, K)]], rows2.at[slot],
                         gsems[slot])

    def wait_gather(slot, c):
        pltpu.make_async_copy(hs.at[sbig.at[pl.ds(c * K, K)]],
                              rows2.at[slot], gsems[slot]).wait()

    def fire_scatter(slot):
        pltpu.async_copy(rows2.at[slot], agg_sh.at[didx2.at[slot]],
                         ssems[slot], add=True)

    def wait_scatter(slot):
        pltpu.make_async_copy(rows2.at[slot], agg_sh.at[didx2.at[slot]],
                              ssems[slot]).wait()

    def scale(slot, c):
        def gbody(g, _):
            w16 = wbig[pl.ds(c * K + g * LANE, LANE)]
            for jj in range(LANE):
                wv = w16.at[jnp.full((LANE,), jj, i32)].get(
                    mode="promise_in_bounds")
                row = g * LANE + jj
                for sbl in range(D // LANE):
                    col = sbl * LANE
                    rows2[slot, row, pl.ds(col, LANE)] = (
                        rows2[slot, row, pl.ds(col, LANE)] * wv)
            return 0

        lax.fori_loop(0, K // LANE, gbody, 0)

    fire_didx(0, 0)
    fire_gather(0, 0)
    wait_gather(0, 0)
    fire_didx(1, 1)
    fire_gather(1, 1)
    scale(0, 0)
    wait_didx(0, 0)
    fire_scatter(0)

    def pair(cc, _):
        c1 = 2 * cc + 1
        wait_gather(1, c1)
        wait_scatter(0)
        fire_didx(0, c1 + 1)
        fire_gather(0, c1 + 1)
        scale(1, c1)
        wait_didx(1, c1)
        fire_scatter(1)

        c2 = c1 + 1
        wait_gather(0, c2)
        wait_scatter(1)

        @pl.when(c2 < ROWS_PER_W - 1)
        def _():
            fire_didx(1, c2 + 1)
            fire_gather(1, c2 + 1)
        scale(0, c2)
        wait_didx(0, c2)
        fire_scatter(0)
        return 0

    lax.fori_loop(0, (ROWS_PER_W - 1) // 2, pair, 0)
    wait_scatter(0)
    plsc.subcore_barrier()

    def obody(j, _):
        i = sid + NS * j

        @pl.when(i < NCHUNK)
        def _():
            pltpu.sync_copy(agg_sh.at[pl.ds(i * K, K)],
                            out.at[cid, pl.ds(i * K, K)])
        return 0

    lax.fori_loop(0, (NCHUNK + NS - 1) // NS, obody, 0)


_agg_call = functools.partial(
    pl.kernel, _agg_body,
    out_type=jax.ShapeDtypeStruct((NC, N, D), f32),
    mesh=_MESH,
    scratch_types=[
        pltpu.VMEM((PER_W,), i32), pltpu.VMEM((PER_W,), f32),
        pltpu.VMEM((2, K), i32),
        pltpu.VMEM((2, K, D), f32),
        pltpu.VMEM_SHARED((N, D), f32),
        pltpu.SemaphoreType.DMA, pltpu.SemaphoreType.DMA,
        pltpu.SemaphoreType.DMA, pltpu.SemaphoreType.DMA,
        pltpu.SemaphoreType.DMA, pltpu.SemaphoreType.DMA,
    ],
)()


def _prep_body(degs_ref, rs_ref, scl_ref):
    d = degs_ref[0] + degs_ref[1]
    dout = d[0:3]
    din = d[3:6]
    cnt = d[6:9]
    rs_ref[...] = lax.rsqrt(jnp.where(dout > 0, dout, 1.0))
    scl_ref[...] = (lax.rsqrt(jnp.where(din > 0, din, 1.0))
                    / jnp.maximum(cnt, 1.0))


def _xscale_body(x_ref, rs_ref, out_ref):
    out_ref[...] = x_ref[...] * rs_ref[...]


def _layer_body(aggp_ref, scl_ref, rsn_ref, w_ref, b_ref, out_ref):
    a = aggp_ref[0] + aggp_ref[1]
    h = a * scl_ref[...]
    h = jnp.dot(h, w_ref[...], preferred_element_type=f32) + b_ref[...]
    out_ref[...] = h * rsn_ref[...]


_R = 2000


def _layer_call(aggp, scl_col, rsn_col, w, b_row):
    return pl.pallas_call(
        _layer_body,
        out_shape=jax.ShapeDtypeStruct((N, D), f32),
        grid=(N // _R,),
        in_specs=[
            pl.BlockSpec((NC, _R, D), lambda i: (0, i, 0)),
            pl.BlockSpec((_R, 1), lambda i: (i, 0)),
            pl.BlockSpec((_R, 1), lambda i: (i, 0)),
            pl.BlockSpec((D, D), lambda i: (0, 0)),
            pl.BlockSpec((1, D), lambda i: (0, 0)),
        ],
        out_specs=pl.BlockSpec((_R, D), lambda i: (i, 0)),
    )(aggp, scl_col, rsn_col, w, b_row)


def kernel(x, edge_index0, edge_index1, edge_index2, ew0, ew1, ew2,
           W1, b1, W2, b2, W3, b3):
    srcs = [ei[0] for ei in (edge_index0, edge_index1, edge_index2)]
    dsts = [ei[1] for ei in (edge_index0, edge_index1, edge_index2)]
    ews = (ew0, ew1, ew2)

    degs = _deg_call(srcs[0], dsts[0], ews[0],
                     srcs[1], dsts[1], ews[1],
                     srcs[2], dsts[2], ews[2])
    degs = degs.reshape(NC, 9, NP)

    rs3, scl3 = pl.pallas_call(
        _prep_body,
        out_shape=(jax.ShapeDtypeStruct((3, NP), f32),
                   jax.ShapeDtypeStruct((3, NP), f32)),
    )(degs)

    rs_cols = [rs3[g, :N].reshape(N, 1) for g in range(3)]
    scl_cols = [scl3[g, :N].reshape(N, 1) for g in range(3)]
    ones_col = jnp.ones((N, 1), f32)

    h = pl.pallas_call(
        _xscale_body,
        out_shape=jax.ShapeDtypeStruct((N, D), f32),
    )(x, rs_cols[0])

    weights = ((W1, b1), (W2, b2), (W3, b3))
    for i, (w, b) in enumerate(weights):
        aggp = _agg_call(h, srcs[i], dsts[i], ews[i])
        rsn = rs_cols[i + 1] if i < 2 else ones_col
        h = _layer_call(aggp, scl_cols[i], rsn, w, b.reshape(1, D))
    return h


# speedup vs baseline: 31.9848x; 1.1038x over previous
import functools

import jax
import jax.numpy as jnp
from jax import lax
from jax.experimental import pallas as pl
from jax.experimental.pallas import tpu as pltpu
from jax.experimental.pallas import tpu_sc as plsc

f32 = jnp.float32
i32 = jnp.int32

NC = 2
NS = 16
NW = NC * NS
LANE = 16
K = 80
CHE = 2000

N = 10000
E = 320000
D = 128
NP = 10240

NCHUNK = N // K
NCHUNKP = NP // K
PER_W = E // NW
ROWS_PER_W = PER_W // K
_MESH = plsc.VectorSubcoreMesh(core_axis_name="c", subcore_axis_name="s")


KA = 128
NA = PER_W // KA
KT = PER_W - NA * KA


def _deg_body(src0, dst0, ew0, src1, dst1, ew1, src2, dst2, ew2, out,
              wbig, sidx2, didx2, sidx_t, didx_t, ones_v, zeros_v,
              do0, do1, do2, di0, di1, di2, ct0, ct1, ct2,
              isem0, isem1, asem0, asem1):
    cid = lax.axis_index("c")
    sid = lax.axis_index("s")
    wid = sid * NC + cid
    base_w = wid * PER_W
    ones16 = jnp.ones((LANE,), f32)
    zeros16 = jnp.zeros((LANE,), f32)
    for t in range(KA // LANE):
        ones_v[pl.ds(t * LANE, LANE)] = ones16
    for t in range(8 * K // LANE):
        zeros_v[pl.ds(t * LANE, LANE)] = zeros16
    sh = (do0, do1, do2, di0, di1, di2, ct0, ct1, ct2)
    isems = (isem0, isem1)
    asems = (asem0, asem1)

    for a in range(9):
        pltpu.sync_copy(zeros_v, sh[a].at[pl.ds(sid * 8 * K, 8 * K)])
    plsc.subcore_barrier()

    graphs = ((src0, dst0, ew0, do0, di0, ct0),
              (src1, dst1, ew1, do1, di1, ct1),
              (src2, dst2, ew2, do2, di2, ct2))
    for (sg, dg, wg, dog, dig, ctg) in graphs:
        pltpu.sync_copy(wg.at[pl.ds(base_w, PER_W)], wbig)

        def fire_idx(slot, c, sg=sg, dg=dg):
            pltpu.async_copy(sg.at[pl.ds(base_w + c * KA, KA)],
                             sidx2.at[slot], isems[slot])
            pltpu.async_copy(dg.at[pl.ds(base_w + c * KA, KA)],
                             didx2.at[slot], isems[slot])

        def wait_idx(slot, c, sg=sg, dg=dg):
            pltpu.make_async_copy(sg.at[pl.ds(base_w + c * KA, KA)],
                                  sidx2.at[slot], isems[slot]).wait()
            pltpu.make_async_copy(dg.at[pl.ds(base_w + c * KA, KA)],
                                  didx2.at[slot], isems[slot]).wait()

        def fire_scats(slot, c, dog=dog, dig=dig, ctg=ctg):
            w = wbig.at[pl.ds(c * KA, KA)]
            pltpu.async_copy(w, dog.at[sidx2.at[slot]], asems[slot],
                             add=True)
            pltpu.async_copy(w, dig.at[didx2.at[slot]], asems[slot],
                             add=True)
            pltpu.async_copy(ones_v, ctg.at[didx2.at[slot]], asems[slot],
                             add=True)

        def wait_scats(slot, dog=dog, dig=dig, ctg=ctg):
            pltpu.make_async_copy(wbig.at[pl.ds(0, KA)],
                                  dog.at[sidx2.at[slot]],
                                  asems[slot]).wait()
            pltpu.make_async_copy(wbig.at[pl.ds(0, KA)],
                                  dig.at[didx2.at[slot]],
                                  asems[slot]).wait()
            pltpu.make_async_copy(ones_v, ctg.at[didx2.at[slot]],
                                  asems[slot]).wait()

        fire_idx(0, 0)
        wait_idx(0, 0)
        fire_idx(1, 1)
        fire_scats(0, 0)

        def pair(cc, _, fire_idx=fire_idx, wait_idx=wait_idx,
                 fire_scats=fire_scats, wait_scats=wait_scats):
            c1 = 2 * cc + 1
            wait_idx(1, c1)
            wait_scats(0)
            fire_idx(0, c1 + 1)
            fire_scats(1, c1)

            c2 = c1 + 1
            wait_idx(0, c2)
            wait_scats(1)

            @pl.when(c2 < NA - 1)
            def _():
                fire_idx(1, c2 + 1)
            fire_scats(0, c2)
            return 0

        lax.fori_loop(0, (NA - 1) // 2, pair, 0)
        cl = NA - 1
        wait_idx(1, cl)
        wait_scats(0)
        fire_scats(1, cl)
        wait_scats(1)

        toff = base_w + NA * KA
        pltpu.sync_copy(sg.at[pl.ds(toff, KT)], sidx_t)
        pltpu.sync_copy(dg.at[pl.ds(toff, KT)], didx_t)
        pltpu.sync_copy(wbig.at[pl.ds(NA * KA, KT)], dog.at[sidx_t],
                        add=True)
        pltpu.sync_copy(wbig.at[pl.ds(NA * KA, KT)], dig.at[didx_t],
                        add=True)
        pltpu.sync_copy(ones_v.at[pl.ds(0, KT)], ctg.at[didx_t], add=True)
    plsc.subcore_barrier()

    for a in range(9):
        pltpu.sync_copy(sh[a].at[pl.ds(sid * 8 * K, 8 * K)],
                        out.at[cid, 0, pl.ds(a * NP + sid * 8 * K, 8 * K)])


_deg_call = functools.partial(
    pl.kernel, _deg_body,
    out_type=jax.ShapeDtypeStruct((NC, 1, 9 * NP), f32),
    mesh=_MESH,
    scratch_types=(
        [pltpu.VMEM((PER_W,), f32),
         pltpu.VMEM((2, KA), i32), pltpu.VMEM((2, KA), i32),
         pltpu.VMEM((KT,), i32), pltpu.VMEM((KT,), i32),
         pltpu.VMEM((KA,), f32), pltpu.VMEM((8 * K,), f32)]
        + [pltpu.VMEM_SHARED((NP,), f32)] * 9
        + [pltpu.SemaphoreType.DMA] * 4
    ),
)()


KC = 128
NAC = PER_W // KC
NZC = N // KC
ZT = NZC * KC


def _agg_body(hs, srcg, dstg, ewg, out,
              sbig, wring, didx2, didx_t, wtail, rows2, agg_sh,
              gsem0, gsem1, ssem0, ssem1, dsem0, dsem1):
    cid = lax.axis_index("c")
    sid = lax.axis_index("s")
    wid = sid * NC + cid
    base_w = wid * PER_W
    zeros16 = jnp.zeros((LANE,), f32)

    pltpu.sync_copy(srcg.at[pl.ds(base_w, PER_W)], sbig)

    def zrow(r, _):
        for sbl in range(D // LANE):
            rows2[0, r, pl.ds(sbl * LANE, LANE)] = zeros16
        return 0

    lax.fori_loop(0, KC, zrow, 0)

    def zbody(j, _):
        i = sid + NS * j

        @pl.when(i < NZC)
        def _():
            pltpu.sync_copy(rows2.at[0], agg_sh.at[pl.ds(i * KC, KC)])
        return 0

    lax.fori_loop(0, (NZC + NS - 1) // NS, zbody, 0)

    @pl.when(sid == 0)
    def _():
        pltpu.sync_copy(rows2.at[0, pl.ds(0, KT)],
                        agg_sh.at[pl.ds(ZT, KT)])
    plsc.subcore_barrier()

    gsems = (gsem0, gsem1)
    ssems = (ssem0, ssem1)
    dsems = (dsem0, dsem1)

    def fire_ewdidx(slot, c):
        pltpu.async_copy(dstg.at[pl.ds(base_w + c * KC, KC)],
                         didx2.at[slot], dsems[slot])
        pltpu.async_copy(ewg.at[pl.ds(base_w + c * KC, KC)],
                         wring.at[slot], dsems[slot])

    def wait_ewdidx(slot, c):
        pltpu.make_async_copy(dstg.at[pl.ds(base_w + c * KC, KC)],
                              didx2.at[slot], dsems[slot]).wait()
        pltpu.make_async_copy(ewg.at[pl.ds(base_w + c * KC, KC)],
                              wring.at[slot], dsems[slot]).wait()

    def fire_gather(slot, c):
        pltpu.async_copy(hs.at[sbig.at[pl.ds(c * KC, KC)]], rows2.at[slot],
                         gsems[slot])

    def wait_gather(slot, c):
        pltpu.make_async_copy(hs.at[sbig.at[pl.ds(c * KC, KC)]],
                              rows2.at[slot], gsems[slot]).wait()

    def fire_scatter(slot):
        pltpu.async_copy(rows2.at[slot], agg_sh.at[didx2.at[slot]],
                         ssems[slot], add=True)

    def wait_scatter(slot):
        pltpu.make_async_copy(rows2.at[slot], agg_sh.at[didx2.at[slot]],
                              ssems[slot]).wait()

    def scale(slot, c):
        del c

        def gbody(g, _):
            w16 = wring[slot, pl.ds(g * LANE, LANE)]
            for jj in range(LANE):
                wv = w16.at[jnp.full((LANE,), jj, i32)].get(
                    mode="promise_in_bounds")
                row = g * LANE + jj
                for sbl in range(D // LANE):
                    col = sbl * LANE
                    rows2[slot, row, pl.ds(col, LANE)] = (
                        rows2[slot, row, pl.ds(col, LANE)] * wv)
            return 0

        lax.fori_loop(0, KC // LANE, gbody, 0)

    fire_ewdidx(0, 0)
    fire_gather(0, 0)
    wait_gather(0, 0)
    fire_ewdidx(1, 1)
    fire_gather(1, 1)
    wait_ewdidx(0, 0)
    scale(0, 0)
    fire_scatter(0)

    def pair(cc, _):
        c1 = 2 * cc + 1
        wait_gather(1, c1)
        wait_scatter(0)
        fire_ewdidx(0, c1 + 1)
        fire_gather(0, c1 + 1)
        wait_ewdidx(1, c1)
        scale(1, c1)
        fire_scatter(1)

        c2 = c1 + 1
        wait_gather(0, c2)
        wait_scatter(1)

        @pl.when(c2 < NAC - 1)
        def _():
            fire_ewdidx(1, c2 + 1)
            fire_gather(1, c2 + 1)
        wait_ewdidx(0, c2)
        scale(0, c2)
        fire_scatter(0)
        return 0

    lax.fori_loop(0, (NAC - 1) // 2, pair, 0)
    cl = NAC - 1
    wait_gather(1, cl)
    wait_scatter(0)
    wait_ewdidx(1, cl)
    scale(1, cl)
    fire_scatter(1)
    wait_scatter(1)

    toff = base_w + NAC * KC
    pltpu.sync_copy(dstg.at[pl.ds(toff, KT)], didx_t)
    pltpu.sync_copy(ewg.at[pl.ds(toff, KT)], wtail)
    pltpu.async_copy(hs.at[sbig.at[pl.ds(NAC * KC, KT)]],
                     rows2.at[0, pl.ds(0, KT)], gsem0).wait()
    wt = wtail[...]
    for jj in range(KT):
        wv = wt.at[jnp.full((LANE,), jj, i32)].get(mode="promise_in_bounds")
        for sbl in range(D // LANE):
            col = sbl * LANE
            rows2[0, jj, pl.ds(col, LANE)] = (
                rows2[0, jj, pl.ds(col, LANE)] * wv)
    pltpu.sync_copy(rows2.at[0, pl.ds(0, KT)], agg_sh.at[didx_t], add=True)
    plsc.subcore_barrier()

    def obody(j, _):
        i = sid + NS * j

        @pl.when(i < NZC)
        def _():
            pltpu.sync_copy(agg_sh.at[pl.ds(i * KC, KC)],
                            out.at[cid, pl.ds(i * KC, KC)])
        return 0

    lax.fori_loop(0, (NZC + NS - 1) // NS, obody, 0)

    @pl.when(sid == 0)
    def _():
        pltpu.sync_copy(agg_sh.at[pl.ds(ZT, KT)],
                        out.at[cid, pl.ds(ZT, KT)])


_agg_call = functools.partial(
    pl.kernel, _agg_body,
    out_type=jax.ShapeDtypeStruct((NC, N, D), f32),
    mesh=_MESH,
    scratch_types=[
        pltpu.VMEM((PER_W,), i32),
        pltpu.VMEM((2, KC), f32),
        pltpu.VMEM((2, KC), i32),
        pltpu.VMEM((KT,), i32), pltpu.VMEM((KT,), f32),
        pltpu.VMEM((2, KC, D), f32),
        pltpu.VMEM_SHARED((N, D), f32),
        pltpu.SemaphoreType.DMA, pltpu.SemaphoreType.DMA,
        pltpu.SemaphoreType.DMA, pltpu.SemaphoreType.DMA,
        pltpu.SemaphoreType.DMA, pltpu.SemaphoreType.DMA,
    ],
)()


def _prep_body(degs_ref, rs_ref, scl_ref):
    d = degs_ref[0] + degs_ref[1]
    dout = d[0:3]
    din = d[3:6]
    cnt = d[6:9]
    rs_ref[...] = lax.rsqrt(jnp.where(dout > 0, dout, 1.0))
    scl_ref[...] = (lax.rsqrt(jnp.where(din > 0, din, 1.0))
                    / jnp.maximum(cnt, 1.0))


def _xscale_body(x_ref, rs_ref, out_ref):
    out_ref[...] = x_ref[...] * rs_ref[...]


def _layer_body(aggp_ref, scl_ref, rsn_ref, w_ref, b_ref, out_ref):
    a = aggp_ref[0] + aggp_ref[1]
    h = a * scl_ref[...]
    h = jnp.dot(h, w_ref[...], preferred_element_type=f32) + b_ref[...]
    out_ref[...] = h * rsn_ref[...]


_R = 2000


def _layer_call(aggp, scl_col, rsn_col, w, b_row):
    return pl.pallas_call(
        _layer_body,
        out_shape=jax.ShapeDtypeStruct((N, D), f32),
        grid=(N // _R,),
        in_specs=[
            pl.BlockSpec((NC, _R, D), lambda i: (0, i, 0)),
            pl.BlockSpec((_R, 1), lambda i: (i, 0)),
            pl.BlockSpec((_R, 1), lambda i: (i, 0)),
            pl.BlockSpec((D, D), lambda i: (0, 0)),
            pl.BlockSpec((1, D), lambda i: (0, 0)),
        ],
        out_specs=pl.BlockSpec((_R, D), lambda i: (i, 0)),
    )(aggp, scl_col, rsn_col, w, b_row)


def kernel(x, edge_index0, edge_index1, edge_index2, ew0, ew1, ew2,
           W1, b1, W2, b2, W3, b3):
    srcs = [ei[0] for ei in (edge_index0, edge_index1, edge_index2)]
    dsts = [ei[1] for ei in (edge_index0, edge_index1, edge_index2)]
    ews = (ew0, ew1, ew2)

    degs = _deg_call(srcs[0], dsts[0], ews[0],
                     srcs[1], dsts[1], ews[1],
                     srcs[2], dsts[2], ews[2])
    degs = degs.reshape(NC, 9, NP)

    rs3, scl3 = pl.pallas_call(
        _prep_body,
        out_shape=(jax.ShapeDtypeStruct((3, NP), f32),
                   jax.ShapeDtypeStruct((3, NP), f32)),
    )(degs)

    rs_cols = [rs3[g, :N].reshape(N, 1) for g in range(3)]
    scl_cols = [scl3[g, :N].reshape(N, 1) for g in range(3)]
    ones_col = jnp.ones((N, 1), f32)

    h = pl.pallas_call(
        _xscale_body,
        out_shape=jax.ShapeDtypeStruct((N, D), f32),
    )(x, rs_cols[0])

    weights = ((W1, b1), (W2, b2), (W3, b3))
    for i, (w, b) in enumerate(weights):
        aggp = _agg_call(h, srcs[i], dsts[i], ews[i])
        rsn = rs_cols[i + 1] if i < 2 else ones_col
        h = _layer_call(aggp, scl_cols[i], rsn, w, b.reshape(1, D))
    return h


# speedup vs baseline: 34.8115x; 1.0884x over previous
import functools

import jax
import jax.numpy as jnp
from jax import lax
from jax.experimental import pallas as pl
from jax.experimental.pallas import tpu as pltpu
from jax.experimental.pallas import tpu_sc as plsc

f32 = jnp.float32
i32 = jnp.int32

NC = 2
NS = 16
NW = NC * NS
LANE = 16

N = 10000
E = 320000
D = 128
NP = 10240

PER_W = E // NW
KC = 128
NAC = PER_W // KC
KT = PER_W - NAC * KC
NZC = N // KC
ZT = NZC * KC
SEG = NP // NS
_MESH = plsc.VectorSubcoreMesh(core_axis_name="c", subcore_axis_name="s")


def _fill(ref, n, vec16):
    for t in range(n // LANE):
        ref[pl.ds(t * LANE, LANE)] = vec16


def _deg_body(srcg, dstg, ewg, out,
              wbig, sidx2, didx2, sidx_t, didx_t, ones_v, zeros_v,
              dsh0, dsh1, dsh2, isem0, isem1, asem0, asem1):
    cid = lax.axis_index("c")
    sid = lax.axis_index("s")
    wid = sid * NC + cid
    base_w = wid * PER_W
    _fill(ones_v, KC, jnp.ones((LANE,), f32))
    _fill(zeros_v, SEG, jnp.zeros((LANE,), f32))
    sh = (dsh0, dsh1, dsh2)
    isems = (isem0, isem1)
    asems = (asem0, asem1)

    for a in range(3):
        pltpu.sync_copy(zeros_v, sh[a].at[pl.ds(sid * SEG, SEG)])
    plsc.subcore_barrier()

    pltpu.sync_copy(ewg.at[pl.ds(base_w, PER_W)], wbig)

    def fire_idx(slot, c):
        pltpu.async_copy(srcg.at[pl.ds(base_w + c * KC, KC)],
                         sidx2.at[slot], isems[slot])
        pltpu.async_copy(dstg.at[pl.ds(base_w + c * KC, KC)],
                         didx2.at[slot], isems[slot])

    def wait_idx(slot, c):
        pltpu.make_async_copy(srcg.at[pl.ds(base_w + c * KC, KC)],
                              sidx2.at[slot], isems[slot]).wait()
        pltpu.make_async_copy(dstg.at[pl.ds(base_w + c * KC, KC)],
                              didx2.at[slot], isems[slot]).wait()

    def fire_scats(slot, c):
        w = wbig.at[pl.ds(c * KC, KC)]
        pltpu.async_copy(w, dsh0.at[sidx2.at[slot]], asems[slot], add=True)
        pltpu.async_copy(w, dsh1.at[didx2.at[slot]], asems[slot], add=True)
        pltpu.async_copy(ones_v, dsh2.at[didx2.at[slot]], asems[slot],
                         add=True)

    def wait_scats(slot):
        pltpu.make_async_copy(wbig.at[pl.ds(0, KC)], dsh0.at[sidx2.at[slot]],
                              asems[slot]).wait()
        pltpu.make_async_copy(wbig.at[pl.ds(0, KC)], dsh1.at[didx2.at[slot]],
                              asems[slot]).wait()
        pltpu.make_async_copy(ones_v, dsh2.at[didx2.at[slot]],
                              asems[slot]).wait()

    fire_idx(0, 0)
    wait_idx(0, 0)
    fire_idx(1, 1)
    fire_scats(0, 0)

    def pair(cc, _):
        c1 = 2 * cc + 1
        wait_idx(1, c1)
        wait_scats(0)
        fire_idx(0, c1 + 1)
        fire_scats(1, c1)

        c2 = c1 + 1
        wait_idx(0, c2)
        wait_scats(1)

        @pl.when(c2 < NAC - 1)
        def _():
            fire_idx(1, c2 + 1)
        fire_scats(0, c2)
        return 0

    lax.fori_loop(0, (NAC - 1) // 2, pair, 0)
    cl = NAC - 1
    wait_idx(1, cl)
    wait_scats(0)
    fire_scats(1, cl)
    wait_scats(1)

    toff = base_w + NAC * KC
    pltpu.sync_copy(srcg.at[pl.ds(toff, KT)], sidx_t)
    pltpu.sync_copy(dstg.at[pl.ds(toff, KT)], didx_t)
    pltpu.sync_copy(wbig.at[pl.ds(NAC * KC, KT)], dsh0.at[sidx_t], add=True)
    pltpu.sync_copy(wbig.at[pl.ds(NAC * KC, KT)], dsh1.at[didx_t], add=True)
    pltpu.sync_copy(ones_v.at[pl.ds(0, KT)], dsh2.at[didx_t], add=True)
    plsc.subcore_barrier()

    for a in range(3):
        pltpu.sync_copy(sh[a].at[pl.ds(sid * SEG, SEG)],
                        out.at[cid, 0, pl.ds(a * NP + sid * SEG, SEG)])


_deg_call = functools.partial(
    pl.kernel, _deg_body,
    out_type=jax.ShapeDtypeStruct((NC, 1, 3 * NP), f32),
    mesh=_MESH,
    scratch_types=(
        [pltpu.VMEM((PER_W,), f32),
         pltpu.VMEM((2, KC), i32), pltpu.VMEM((2, KC), i32),
         pltpu.VMEM((KT,), i32), pltpu.VMEM((KT,), i32),
         pltpu.VMEM((KC,), f32), pltpu.VMEM((SEG,), f32)]
        + [pltpu.VMEM_SHARED((NP,), f32)] * 3
        + [pltpu.SemaphoreType.DMA] * 4
    ),
)()


def _make_agg(with_deg):
    def body(*refs):
        if with_deg:
            (hs, srcg, dstg, ewg, srcd, dstd, ewd, out, dout,
             sbig, wring, didx2, didx_t, wtail, rows2, agg_sh,
             gsem0, gsem1, ssem0, ssem1, dsem0, dsem1,
             dsidx2, ddidx2, dwring, ones_v, dzeros,
             dsidx_t, ddidx_t, dwtail, dsh0, dsh1, dsh2,
             disem0, disem1, dasem0, dasem1) = refs
        else:
            (hs, srcg, dstg, ewg, out,
             sbig, wring, didx2, didx_t, wtail, rows2, agg_sh,
             gsem0, gsem1, ssem0, ssem1, dsem0, dsem1) = refs
        cid = lax.axis_index("c")
        sid = lax.axis_index("s")
        wid = sid * NC + cid
        base_w = wid * PER_W
        zeros16 = jnp.zeros((LANE,), f32)

        pltpu.sync_copy(srcg.at[pl.ds(base_w, PER_W)], sbig)

        def zrow(r, _):
            for sbl in range(D // LANE):
                rows2[0, r, pl.ds(sbl * LANE, LANE)] = zeros16
            return 0

        lax.fori_loop(0, KC, zrow, 0)

        def zbody(j, _):
            i = sid + NS * j

            @pl.when(i < NZC)
            def _():
                pltpu.sync_copy(rows2.at[0], agg_sh.at[pl.ds(i * KC, KC)])
            return 0

        lax.fori_loop(0, (NZC + NS - 1) // NS, zbody, 0)

        @pl.when(sid == 0)
        def _():
            pltpu.sync_copy(rows2.at[0, pl.ds(0, KT)],
                            agg_sh.at[pl.ds(ZT, KT)])

        if with_deg:
            _fill(ones_v, KC, jnp.ones((LANE,), f32))
            _fill(dzeros, SEG, zeros16)
            dsh = (dsh0, dsh1, dsh2)
            for a in range(3):
                pltpu.sync_copy(dzeros, dsh[a].at[pl.ds(sid * SEG, SEG)])
        plsc.subcore_barrier()

        gsems = (gsem0, gsem1)
        ssems = (ssem0, ssem1)
        dsems = (dsem0, dsem1)

        def fire_ewdidx(slot, c):
            pltpu.async_copy(dstg.at[pl.ds(base_w + c * KC, KC)],
                             didx2.at[slot], dsems[slot])
            pltpu.async_copy(ewg.at[pl.ds(base_w + c * KC, KC)],
                             wring.at[slot], dsems[slot])

        def wait_ewdidx(slot, c):
            pltpu.make_async_copy(dstg.at[pl.ds(base_w + c * KC, KC)],
                                  didx2.at[slot], dsems[slot]).wait()
            pltpu.make_async_copy(ewg.at[pl.ds(base_w + c * KC, KC)],
                                  wring.at[slot], dsems[slot]).wait()

        def fire_gather(slot, c):
            pltpu.async_copy(hs.at[sbig.at[pl.ds(c * KC, KC)]],
                             rows2.at[slot], gsems[slot])

        def wait_gather(slot, c):
            pltpu.make_async_copy(hs.at[sbig.at[pl.ds(c * KC, KC)]],
                                  rows2.at[slot], gsems[slot]).wait()

        def fire_scatter(slot):
            pltpu.async_copy(rows2.at[slot], agg_sh.at[didx2.at[slot]],
                             ssems[slot], add=True)

        def wait_scatter(slot):
            pltpu.make_async_copy(rows2.at[slot], agg_sh.at[didx2.at[slot]],
                                  ssems[slot]).wait()

        def scale(slot):
            def gbody(g, _):
                w16 = wring[slot, pl.ds(g * LANE, LANE)]
                for jj in range(LANE):
                    wv = w16.at[jnp.full((LANE,), jj, i32)].get(
                        mode="promise_in_bounds")
                    row = g * LANE + jj
                    for sbl in range(D // LANE):
                        col = sbl * LANE
                        rows2[slot, row, pl.ds(col, LANE)] = (
                            rows2[slot, row, pl.ds(col, LANE)] * wv)
                return 0

            lax.fori_loop(0, KC // LANE, gbody, 0)

        if with_deg:
            disems = (disem0, disem1)
            dasems = (dasem0, dasem1)

            def dfire_idx(slot, c):
                pltpu.async_copy(srcd.at[pl.ds(base_w + c * KC, KC)],
                                 dsidx2.at[slot], disems[slot])
                pltpu.async_copy(dstd.at[pl.ds(base_w + c * KC, KC)],
                                 ddidx2.at[slot], disems[slot])
                pltpu.async_copy(ewd.at[pl.ds(base_w + c * KC, KC)],
                                 dwring.at[slot], disems[slot])

            def dwait_idx(slot, c):
                pltpu.make_async_copy(srcd.at[pl.ds(base_w + c * KC, KC)],
                                      dsidx2.at[slot], disems[slot]).wait()
                pltpu.make_async_copy(dstd.at[pl.ds(base_w + c * KC, KC)],
                                      ddidx2.at[slot], disems[slot]).wait()
                pltpu.make_async_copy(ewd.at[pl.ds(base_w + c * KC, KC)],
                                      dwring.at[slot], disems[slot]).wait()

            def dfire_scats(slot):
                w = dwring.at[slot]
                pltpu.async_copy(w, dsh0.at[dsidx2.at[slot]], dasems[slot],
                                 add=True)
                pltpu.async_copy(w, dsh1.at[ddidx2.at[slot]], dasems[slot],
                                 add=True)
                pltpu.async_copy(ones_v, dsh2.at[ddidx2.at[slot]],
                                 dasems[slot], add=True)

            def dwait_scats(slot):
                pltpu.make_async_copy(dwring.at[slot],
                                      dsh0.at[dsidx2.at[slot]],
                                      dasems[slot]).wait()
                pltpu.make_async_copy(dwring.at[slot],
                                      dsh1.at[ddidx2.at[slot]],
                                      dasems[slot]).wait()
                pltpu.make_async_copy(ones_v, dsh2.at[ddidx2.at[slot]],
                                      dasems[slot]).wait()

        fire_ewdidx(0, 0)
        fire_gather(0, 0)
        if with_deg:
            dfire_idx(0, 0)
        wait_gather(0, 0)
        fire_ewdidx(1, 1)
        fire_gather(1, 1)
        if with_deg:
            dwait_idx(0, 0)
            dfire_idx(1, 1)
            dfire_scats(0)
        wait_ewdidx(0, 0)
        scale(0)
        fire_scatter(0)

        def pair(cc, _):
            c1 = 2 * cc + 1
            wait_gather(1, c1)
            wait_scatter(0)
            fire_ewdidx(0, c1 + 1)
            fire_gather(0, c1 + 1)
            if with_deg:
                dwait_idx(1, c1)
                dwait_scats(0)
                dfire_idx(0, c1 + 1)
                dfire_scats(1)
            wait_ewdidx(1, c1)
            scale(1)
            fire_scatter(1)

            c2 = c1 + 1
            wait_gather(0, c2)
            wait_scatter(1)

            @pl.when(c2 < NAC - 1)
            def _():
                fire_ewdidx(1, c2 + 1)
                fire_gather(1, c2 + 1)
            if with_deg:
                dwait_idx(0, c2)
                dwait_scats(1)

                @pl.when(c2 < NAC - 1)
                def _():
                    dfire_idx(1, c2 + 1)
                dfire_scats(0)
            wait_ewdidx(0, c2)
            scale(0)
            fire_scatter(0)
            return 0

        lax.fori_loop(0, (NAC - 1) // 2, pair, 0)
        cl = NAC - 1
        wait_gather(1, cl)
        wait_scatter(0)
        if with_deg:
            dwait_idx(1, cl)
            dwait_scats(0)
            dfire_scats(1)
        wait_ewdidx(1, cl)
        scale(1)
        fire_scatter(1)
        wait_scatter(1)
        if with_deg:
            dwait_scats(1)

        toff = base_w + NAC * KC
        pltpu.sync_copy(dstg.at[pl.ds(toff, KT)], didx_t)
        pltpu.sync_copy(ewg.at[pl.ds(toff, KT)], wtail)
        pltpu.async_copy(hs.at[sbig.at[pl.ds(NAC * KC, KT)]],
                         rows2.at[0, pl.ds(0, KT)], gsem0).wait()
        wt = wtail[...]
        for jj in range(KT):
            wv = wt.at[jnp.full((LANE,), jj, i32)].get(
                mode="promise_in_bounds")
            for sbl in range(D // LANE):
                col = sbl * LANE
                rows2[0, jj, pl.ds(col, LANE)] = (
                    rows2[0, jj, pl.ds(col, LANE)] * wv)
        pltpu.sync_copy(rows2.at[0, pl.ds(0, KT)], agg_sh.at[didx_t],
                        add=True)
        if with_deg:
            pltpu.sync_copy(srcd.at[pl.ds(toff, KT)], dsidx_t)
            pltpu.sync_copy(dstd.at[pl.ds(toff, KT)], ddidx_t)
            pltpu.sync_copy(ewd.at[pl.ds(toff, KT)], dwtail)
            pltpu.sync_copy(dwtail, dsh0.at[dsidx_t], add=True)
            pltpu.sync_copy(dwtail, dsh1.at[ddidx_t], add=True)
            pltpu.sync_copy(ones_v.at[pl.ds(0, KT)], dsh2.at[ddidx_t],
                            add=True)
        plsc.subcore_barrier()

        def obody(j, _):
            i = sid + NS * j

            @pl.when(i < NZC)
            def _():
                pltpu.sync_copy(agg_sh.at[pl.ds(i * KC, KC)],
                                out.at[cid, pl.ds(i * KC, KC)])
            return 0

        lax.fori_loop(0, (NZC + NS - 1) // NS, obody, 0)

        @pl.when(sid == 0)
        def _():
            pltpu.sync_copy(agg_sh.at[pl.ds(ZT, KT)],
                            out.at[cid, pl.ds(ZT, KT)])

        if with_deg:
            for a in range(3):
                pltpu.sync_copy(
                    dsh[a].at[pl.ds(sid * SEG, SEG)],
                    dout.at[cid, 0, pl.ds(a * NP + sid * SEG, SEG)])

    out_type = jax.ShapeDtypeStruct((NC, N, D), f32)
    scratch = [
        pltpu.VMEM((PER_W,), i32),
        pltpu.VMEM((2, KC), f32),
        pltpu.VMEM((2, KC), i32),
        pltpu.VMEM((KT,), i32), pltpu.VMEM((KT,), f32),
        pltpu.VMEM((2, KC, D), f32),
        pltpu.VMEM_SHARED((N, D), f32),
    ] + [pltpu.SemaphoreType.DMA] * 6
    if with_deg:
        out_type = [out_type, jax.ShapeDtypeStruct((NC, 1, 3 * NP), f32)]
        scratch = scratch + [
            pltpu.VMEM((2, KC), i32), pltpu.VMEM((2, KC), i32),
            pltpu.VMEM((2, KC), f32),
            pltpu.VMEM((KC,), f32), pltpu.VMEM((SEG,), f32),
            pltpu.VMEM((KT,), i32), pltpu.VMEM((KT,), i32),
            pltpu.VMEM((KT,), f32),
        ] + [pltpu.VMEM_SHARED((NP,), f32)] * 3 \
          + [pltpu.SemaphoreType.DMA] * 4
    return functools.partial(
        pl.kernel, body, out_type=out_type, mesh=_MESH,
        scratch_types=scratch)()


_agg_deg_call = _make_agg(True)
_agg_call = _make_agg(False)


def _prep_body(degs_ref, rs_ref):
    d = degs_ref[0] + degs_ref[1]
    dout = d[0:1]
    rs_ref[...] = lax.rsqrt(jnp.where(dout > 0, dout, 1.0))


def _xscale_body(x_ref, rs_ref, out_ref):
    out_ref[...] = x_ref[...] * rs_ref[...]


def _layer_body(aggp_ref, dina, dinb, cnta, cntb, douta, doutb,
                w_ref, b_ref, out_ref):
    a = aggp_ref[0] + aggp_ref[1]
    din = dina[...] + dinb[...]
    cnt = cnta[...] + cntb[...]
    scl = lax.rsqrt(jnp.where(din > 0, din, 1.0)) / jnp.maximum(cnt, 1.0)
    h = a * scl
    h = jnp.dot(h, w_ref[...], preferred_element_type=f32) + b_ref[...]
    dout = douta[...] + doutb[...]
    out_ref[...] = h * lax.rsqrt(jnp.where(dout > 0, dout, 1.0))


_R = 2000


def _layer_call(aggp, din_cols, cnt_cols, dout_cols, w, b_row):
    col_spec = pl.BlockSpec((_R, 1), lambda i: (i, 0))
    return pl.pallas_call(
        _layer_body,
        out_shape=jax.ShapeDtypeStruct((N, D), f32),
        grid=(N // _R,),
        in_specs=[
            pl.BlockSpec((NC, _R, D), lambda i: (0, i, 0)),
            col_spec, col_spec, col_spec, col_spec, col_spec, col_spec,
            pl.BlockSpec((D, D), lambda i: (0, 0)),
            pl.BlockSpec((1, D), lambda i: (0, 0)),
        ],
        out_specs=pl.BlockSpec((_R, D), lambda i: (i, 0)),
    )(aggp, din_cols[0], din_cols[1], cnt_cols[0], cnt_cols[1],
      dout_cols[0], dout_cols[1], w, b_row)


def kernel(x, edge_index0, edge_index1, edge_index2, ew0, ew1, ew2,
           W1, b1, W2, b2, W3, b3):
    srcs = [ei[0] for ei in (edge_index0, edge_index1, edge_index2)]
    dsts = [ei[1] for ei in (edge_index0, edge_index1, edge_index2)]
    ews = (ew0, ew1, ew2)

    degs0 = _deg_call(srcs[0], dsts[0], ews[0])

    def cols(degs, k):
        return (degs[0, 0, k * NP:k * NP + N].reshape(N, 1),
                degs[1, 0, k * NP:k * NP + N].reshape(N, 1))

    rs0 = pl.pallas_call(
        _prep_body,
        out_shape=jax.ShapeDtypeStruct((1, NP), f32),
    )(degs0.reshape(NC, 3, NP))
    rs0_col = rs0[0, :N].reshape(N, 1)

    xs = pl.pallas_call(
        _xscale_body,
        out_shape=jax.ShapeDtypeStruct((N, D), f32),
    )(x, rs0_col)

    ones_col = jnp.ones((N, 1), f32)
    zeros_col = jnp.zeros((N, 1), f32)

    aggp0, degs1 = _agg_deg_call(xs, srcs[0], dsts[0], ews[0],
                                 srcs[1], dsts[1], ews[1])
    h1 = _layer_call(aggp0, cols(degs0, 1), cols(degs0, 2), cols(degs1, 0),
                     W1, b1.reshape(1, D))

    aggp1, degs2 = _agg_deg_call(h1, srcs[1], dsts[1], ews[1],
                                 srcs[2], dsts[2], ews[2])
    h2 = _layer_call(aggp1, cols(degs1, 1), cols(degs1, 2), cols(degs2, 0),
                     W2, b2.reshape(1, D))

    aggp2 = _agg_call(h2, srcs[2], dsts[2], ews[2])
    res = _layer_call(aggp2, cols(degs2, 1), cols(degs2, 2),
                      (ones_col, zeros_col), W3, b3.reshape(1, D))
    return res
